# baseline (speedup 1.0000x reference)
import functools

import jax
import jax.numpy as jnp
from jax import lax
from jax.experimental import pallas as pl
from jax.experimental.pallas import tpu as pltpu
from jax.experimental.pallas import tpu_sc as plsc

_DIM = 32
_BATCH = 16384
_A = 1.0
_B = 1.0

_info = plsc.get_sparse_core_info()
_NC, _NS, _L = _info.num_cores, _info.num_subcores, _info.num_lanes
_NW = _NC * _NS
_BPW = _BATCH // _NW
_NGRP = _BPW // _L

_mesh = plsc.VectorSubcoreMesh(core_axis_name="c", subcore_axis_name="s")


@functools.partial(
    pl.kernel,
    mesh=_mesh,
    compiler_params=pltpu.CompilerParams(
        needs_layout_passes=False, use_tc_tiling_on_sc=False
    ),
    out_type=[
        jax.ShapeDtypeStruct((_NW * _L,), jnp.float32),
        jax.ShapeDtypeStruct((_NW * _L,), jnp.float32),
    ],
    scratch_types=[
        pltpu.VMEM((_BPW,), jnp.int32),
        pltpu.VMEM((_BPW,), jnp.int32),
        pltpu.VMEM((_BPW,), jnp.float32),
        pltpu.VMEM((_DIM * _BPW,), jnp.float32),
        pltpu.VMEM((_DIM * _BPW,), jnp.float32),
        pltpu.VMEM((_BPW,), jnp.float32),
        pltpu.VMEM((_L,), jnp.float32),
        pltpu.VMEM((_L,), jnp.float32),
        pltpu.SemaphoreType.DMA,
        pltpu.SemaphoreType.DMA,
        pltpu.SemaphoreType.DMA,
    ],
)
def _wrmf_sc(uid_hbm, iid_hbm, lab_hbm, ut_hbm, it_hbm, bt_hbm,
             loss_out, l2_out,
             uid_v, iid_v, lab_v, uval_v, ival_v, bias_v,
             loss_st, l2_st, sem_u, sem_i, sem_b):
    wid = lax.axis_index("s") * _NC + lax.axis_index("c")
    base = wid * _BPW

    pltpu.sync_copy(uid_hbm.at[pl.ds(base, _BPW)], uid_v)
    pltpu.sync_copy(iid_hbm.at[pl.ds(base, _BPW)], iid_v)
    pltpu.sync_copy(lab_hbm.at[pl.ds(base, _BPW)], lab_v)

    copies = []
    for d in range(_DIM):
        copies.append(pltpu.async_copy(
            ut_hbm.at[d].at[uid_v], uval_v.at[pl.ds(d * _BPW, _BPW)], sem_u))
        copies.append(pltpu.async_copy(
            it_hbm.at[d].at[iid_v], ival_v.at[pl.ds(d * _BPW, _BPW)], sem_i))
    copies.append(pltpu.async_copy(bt_hbm.at[iid_v], bias_v, sem_b))
    for cp in copies:
        cp.wait()

    def body(g, carry):
        loss_acc, l2_acc = carry
        goff = g * _L
        acc = jnp.zeros((_L,), jnp.float32)
        sq = jnp.zeros((_L,), jnp.float32)
        for d in range(_DIM):
            uu = uval_v[pl.ds(d * _BPW + goff, _L)]
            ii = ival_v[pl.ds(d * _BPW + goff, _L)]
            acc = acc + uu * ii
            sq = sq + (uu * uu + ii * ii)
        lab = lab_v[pl.ds(goff, _L)]
        pred = acc + bias_v[pl.ds(goff, _L)]
        w = (_A - _B) * lab + _B
        err = lab - pred
        return loss_acc + w * err * err, l2_acc + sq

    loss_vec, l2_vec = lax.fori_loop(
        0, _NGRP,
        body,
        (jnp.zeros((_L,), jnp.float32), jnp.zeros((_L,), jnp.float32)),
    )

    loss_st[...] = loss_vec
    l2_st[...] = 0.5 * l2_vec
    pltpu.sync_copy(loss_st, loss_out.at[pl.ds(wid * _L, _L)])
    pltpu.sync_copy(l2_st, l2_out.at[pl.ds(wid * _L, _L)])


def kernel(user_id, item_id, label, user_table, item_table, item_bias_table):
    loss_p, l2_p = _wrmf_sc(
        user_id.astype(jnp.int32),
        item_id.astype(jnp.int32),
        label,
        user_table.T,
        item_table.T,
        item_bias_table.reshape(-1),
    )
    return jnp.sum(loss_p), jnp.sum(l2_p)


# speedup vs baseline: 5.6680x; 5.6680x over previous
import functools

import jax
import jax.numpy as jnp
from jax import lax
from jax.experimental import pallas as pl
from jax.experimental.pallas import tpu as pltpu
from jax.experimental.pallas import tpu_sc as plsc

_DIM = 32
_BATCH = 16384
_A = 1.0
_B = 1.0

_info = plsc.get_sparse_core_info()
_NC, _NS, _L = _info.num_cores, _info.num_subcores, _info.num_lanes
_NW = _NC * _NS
_BPW = _BATCH // _NW
_NGRP = _BPW // _L

_mesh = plsc.VectorSubcoreMesh(core_axis_name="c", subcore_axis_name="s")


@functools.partial(
    pl.kernel,
    mesh=_mesh,
    compiler_params=pltpu.CompilerParams(
        needs_layout_passes=False, use_tc_tiling_on_sc=False
    ),
    out_type=[
        jax.ShapeDtypeStruct((_NW * _L,), jnp.float32),
        jax.ShapeDtypeStruct((_NW * _L,), jnp.float32),
    ],
    scratch_types=[
        pltpu.VMEM((_BPW,), jnp.int32),
        pltpu.VMEM((_BPW,), jnp.int32),
        pltpu.VMEM((_BPW,), jnp.float32),
        pltpu.VMEM((_BPW, _DIM), jnp.float32),
        pltpu.VMEM((_BPW, _DIM), jnp.float32),
        pltpu.VMEM((_BPW,), jnp.float32),
        pltpu.VMEM((_L,), jnp.float32),
        pltpu.VMEM((_L,), jnp.float32),
        pltpu.SemaphoreType.DMA,
        pltpu.SemaphoreType.DMA,
        pltpu.SemaphoreType.DMA,
    ],
)
def _wrmf_sc(uid_hbm, iid_hbm, lab_hbm, ut_hbm, it_hbm, bt_hbm,
             loss_out, l2_out,
             uid_v, iid_v, lab_v, urows, irows, bias_v,
             loss_st, l2_st, sem_u, sem_i, sem_b):
    wid = lax.axis_index("s") * _NC + lax.axis_index("c")
    base = wid * _BPW

    pltpu.sync_copy(uid_hbm.at[pl.ds(base, _BPW)], uid_v)
    pltpu.sync_copy(iid_hbm.at[pl.ds(base, _BPW)], iid_v)
    pltpu.sync_copy(lab_hbm.at[pl.ds(base, _BPW)], lab_v)

    cp_u = pltpu.async_copy(ut_hbm.at[uid_v], urows, sem_u)
    cp_i = pltpu.async_copy(it_hbm.at[iid_v], irows, sem_i)
    cp_b = pltpu.async_copy(bt_hbm.at[iid_v], bias_v, sem_b)
    cp_u.wait()
    cp_i.wait()
    cp_b.wait()

    lane = lax.broadcasted_iota(jnp.int32, (_L,), 0)

    def body(g, carry):
        loss_acc, l2_acc = carry
        ridx = g * _L + lane
        acc = jnp.zeros((_L,), jnp.float32)
        sq = jnp.zeros((_L,), jnp.float32)
        for d in range(_DIM):
            didx = jnp.full((_L,), d, jnp.int32)
            uu = plsc.load_gather(urows, [ridx, didx])
            ii = plsc.load_gather(irows, [ridx, didx])
            acc = acc + uu * ii
            sq = sq + (uu * uu + ii * ii)
        lab = lab_v[pl.ds(g * _L, _L)]
        pred = acc + bias_v[pl.ds(g * _L, _L)]
        w = (_A - _B) * lab + _B
        err = lab - pred
        return loss_acc + w * err * err, l2_acc + sq

    loss_vec, l2_vec = lax.fori_loop(
        0, _NGRP,
        body,
        (jnp.zeros((_L,), jnp.float32), jnp.zeros((_L,), jnp.float32)),
    )

    loss_st[...] = loss_vec
    l2_st[...] = 0.5 * l2_vec
    pltpu.sync_copy(loss_st, loss_out.at[pl.ds(wid * _L, _L)])
    pltpu.sync_copy(l2_st, l2_out.at[pl.ds(wid * _L, _L)])


def kernel(user_id, item_id, label, user_table, item_table, item_bias_table):
    loss_p, l2_p = _wrmf_sc(
        user_id.astype(jnp.int32),
        item_id.astype(jnp.int32),
        label,
        user_table,
        item_table,
        item_bias_table.reshape(-1),
    )
    return jnp.sum(loss_p), jnp.sum(l2_p)


# speedup vs baseline: 11.2842x; 1.9909x over previous
import functools

import jax
import jax.numpy as jnp
from jax import lax
from jax.experimental import pallas as pl
from jax.experimental.pallas import tpu as pltpu
from jax.experimental.pallas import tpu_sc as plsc

_DIM = 32
_BATCH = 16384
_A = 1.0
_B = 1.0

_info = plsc.get_sparse_core_info()
_NC, _NS, _L = _info.num_cores, _info.num_subcores, _info.num_lanes
_NW = _NC * _NS
_BPW = _BATCH // _NW
_NGRP = _BPW // _L

_mesh = plsc.VectorSubcoreMesh(core_axis_name="c", subcore_axis_name="s")


@functools.partial(
    pl.kernel,
    mesh=_mesh,
    compiler_params=pltpu.CompilerParams(needs_layout_passes=False),
    out_type=[
        jax.ShapeDtypeStruct((_NW * _L,), jnp.float32),
        jax.ShapeDtypeStruct((_NW * _L,), jnp.float32),
    ],
    scratch_types=[
        pltpu.VMEM((_BPW,), jnp.int32),
        pltpu.VMEM((_BPW,), jnp.int32),
        pltpu.VMEM((_BPW,), jnp.float32),
        pltpu.VMEM((_L, 8, 128), jnp.float32),
        pltpu.VMEM((_BPW * _DIM,), jnp.float32),
        pltpu.VMEM((_BPW * _DIM,), jnp.float32),
        pltpu.VMEM((_BPW * 16,), jnp.float32),
        pltpu.VMEM((_L,), jnp.float32),
        pltpu.VMEM((_L,), jnp.float32),
        pltpu.SemaphoreType.DMA,
        pltpu.SemaphoreType.DMA,
    ],
)
def _wrmf_sc(uid_hbm, iid_hbm, lab_hbm, ut_hbm, it_hbm, bt_hbm,
             loss_out, l2_out,
             uid_v, iid_v, lab_v, bundle, uval, ival, brun,
             loss_st, l2_st, sem, sem_b):
    wid = lax.axis_index("s") * _NC + lax.axis_index("c")
    base = wid * _BPW

    pltpu.sync_copy(uid_hbm.at[pl.ds(base, _BPW)], uid_v)
    pltpu.sync_copy(iid_hbm.at[pl.ds(base, _BPW)], iid_v)
    pltpu.sync_copy(lab_hbm.at[pl.ds(base, _BPW)], lab_v)

    lane = lax.broadcasted_iota(jnp.int32, (_L,), 0)

    def fetch_extract(ids_ref, table, dst, c):
        idv = ids_ref[pl.ds(c * _L, _L)]
        low = idv & 127
        ridx = c * _L + lane
        blks = [pl.multiple_of((idv[j] >> 7) << 7, 128) for j in range(_L)]
        for dr in range(_DIM // 8):
            cps = []
            for j in range(_L):
                cps.append(pltpu.async_copy(
                    table.at[pl.ds(dr * 8, 8), pl.ds(blks[j], 128)],
                    bundle.at[j],
                    sem))
            for cp in cps:
                cp.wait()
            for s in range(8):
                d = dr * 8 + s
                val = plsc.load_gather(
                    bundle, [lane, jnp.full((_L,), s, jnp.int32), low])
                plsc.store_scatter(dst, [ridx * _DIM + d], val)

    def chunk(c, carry):
        fetch_extract(uid_v, ut_hbm, uval, c)
        fetch_extract(iid_v, it_hbm, ival, c)
        idv = iid_v[pl.ds(c * _L, _L)]
        run = (idv >> 4) << 4
        for j in range(_L):
            pltpu.async_copy(bt_hbm.at[pl.ds(pl.multiple_of(run[j], 16), 16)],
                             brun.at[pl.ds((c * _L + j) * 16, 16)], sem_b)
        return carry

    lax.fori_loop(0, _NGRP, chunk, 0)
    pltpu.make_async_copy(lab_hbm.at[pl.ds(0, _BPW * 16)], brun, sem_b).wait()

    def body(g, carry):
        loss_acc, l2_acc = carry
        ridx = g * _L + lane
        acc = jnp.zeros((_L,), jnp.float32)
        sq = jnp.zeros((_L,), jnp.float32)
        rbase = ridx * _DIM
        for d in range(_DIM):
            uu = plsc.load_gather(uval, [rbase + d])
            ii = plsc.load_gather(ival, [rbase + d])
            acc = acc + uu * ii
            sq = sq + (uu * uu + ii * ii)
        idv = iid_v[pl.ds(g * _L, _L)]
        bias = plsc.load_gather(brun, [ridx * 16 + (idv & 15)])
        lab = lab_v[pl.ds(g * _L, _L)]
        pred = acc + bias
        w = (_A - _B) * lab + _B
        err = lab - pred
        return loss_acc + w * err * err, l2_acc + sq

    loss_vec, l2_vec = lax.fori_loop(
        0, _NGRP,
        body,
        (jnp.zeros((_L,), jnp.float32), jnp.zeros((_L,), jnp.float32)),
    )

    loss_st[...] = loss_vec
    l2_st[...] = 0.5 * l2_vec
    pltpu.sync_copy(loss_st, loss_out.at[pl.ds(wid * _L, _L)])
    pltpu.sync_copy(l2_st, l2_out.at[pl.ds(wid * _L, _L)])


def kernel(user_id, item_id, label, user_table, item_table, item_bias_table):
    loss_p, l2_p = _wrmf_sc(
        user_id.astype(jnp.int32),
        item_id.astype(jnp.int32),
        label,
        user_table.T,
        item_table.T,
        item_bias_table.reshape(-1),
    )
    return jnp.sum(loss_p), jnp.sum(l2_p)


# speedup vs baseline: 15.1212x; 1.3400x over previous
import functools

import jax
import jax.numpy as jnp
from jax import lax
from jax.experimental import pallas as pl
from jax.experimental.pallas import tpu as pltpu
from jax.experimental.pallas import tpu_sc as plsc

_DIM = 32
_BATCH = 16384
_A = 1.0
_B = 1.0

_info = plsc.get_sparse_core_info()
_NC, _NS, _L = _info.num_cores, _info.num_subcores, _info.num_lanes
_NW = _NC * _NS
_BPW = _BATCH // _NW
_NGRP = _BPW // _L

_mesh = plsc.VectorSubcoreMesh(core_axis_name="c", subcore_axis_name="s")


@functools.partial(
    pl.kernel,
    mesh=_mesh,
    compiler_params=pltpu.CompilerParams(needs_layout_passes=False),
    out_type=[
        jax.ShapeDtypeStruct((_NW * _L,), jnp.float32),
        jax.ShapeDtypeStruct((_NW * _L,), jnp.float32),
    ],
    scratch_types=[
        pltpu.VMEM((_BPW,), jnp.int32),
        pltpu.VMEM((_BPW,), jnp.int32),
        pltpu.VMEM((_BPW,), jnp.float32),
        pltpu.VMEM((_L, 8, 128), jnp.float32),
        pltpu.VMEM((_L, 8, 128), jnp.float32),
        pltpu.VMEM((_BPW * _DIM,), jnp.float32),
        pltpu.VMEM((_BPW * _DIM,), jnp.float32),
        pltpu.VMEM((_BPW * 16,), jnp.float32),
        pltpu.VMEM((_L,), jnp.float32),
        pltpu.VMEM((_L,), jnp.float32),
        pltpu.SemaphoreType.DMA,
        pltpu.SemaphoreType.DMA,
        pltpu.SemaphoreType.DMA,
    ],
)
def _wrmf_sc(uid_hbm, iid_hbm, lab_hbm, ut_hbm, it_hbm, bt_hbm,
             loss_out, l2_out,
             uid_v, iid_v, lab_v, bundle_a, bundle_b, uval, ival, brun,
             loss_st, l2_st, sem_a, sem_b2, sem_b):
    wid = lax.axis_index("s") * _NC + lax.axis_index("c")
    base = wid * _BPW

    pltpu.sync_copy(uid_hbm.at[pl.ds(base, _BPW)], uid_v)
    pltpu.sync_copy(iid_hbm.at[pl.ds(base, _BPW)], iid_v)
    pltpu.sync_copy(lab_hbm.at[pl.ds(base, _BPW)], lab_v)

    lane = lax.broadcasted_iota(jnp.int32, (_L,), 0)

    bundles = (bundle_a, bundle_b)
    sems = (sem_a, sem_b2)

    def chunk(c, carry):
        uvec = uid_v[pl.ds(c * _L, _L)]
        ivec = iid_v[pl.ds(c * _L, _L)]
        ublk = [pl.multiple_of((uvec[j] >> 7) << 7, 128) for j in range(_L)]
        iblk = [pl.multiple_of((ivec[j] >> 7) << 7, 128) for j in range(_L)]
        ulow = uvec & 127
        ilow = ivec & 127
        ridx = c * _L + lane
        phases = ([(ublk, ut_hbm, uval, ulow, dr) for dr in range(4)]
                  + [(iblk, it_hbm, ival, ilow, dr) for dr in range(4)])

        def issue(p):
            blks, table, _, _, dr = phases[p]
            return [pltpu.async_copy(
                table.at[pl.ds(dr * 8, 8), pl.ds(blks[j], 128)],
                bundles[p % 2].at[j], sems[p % 2]) for j in range(_L)]

        cps = [None, None]
        cps[0] = issue(0)
        for p in range(8):
            if p + 1 < 8:
                cps[(p + 1) % 2] = issue(p + 1)
            for cp in cps[p % 2]:
                cp.wait()
            _, _, dst, low, dr = phases[p]
            for s in range(8):
                d = dr * 8 + s
                val = plsc.load_gather(
                    bundles[p % 2],
                    [lane, jnp.full((_L,), s, jnp.int32), low])
                plsc.store_scatter(dst, [ridx * _DIM + d], val)
        run = (ivec >> 4) << 4
        for j in range(_L):
            pltpu.async_copy(bt_hbm.at[pl.ds(pl.multiple_of(run[j], 16), 16)],
                             brun.at[pl.ds((c * _L + j) * 16, 16)], sem_b)
        return carry

    lax.fori_loop(0, _NGRP, chunk, 0)
    pltpu.make_async_copy(lab_hbm.at[pl.ds(0, _BPW * 16)], brun, sem_b).wait()

    def body(g, carry):
        loss_acc, l2_acc = carry
        ridx = g * _L + lane
        acc = jnp.zeros((_L,), jnp.float32)
        sq = jnp.zeros((_L,), jnp.float32)
        rbase = ridx * _DIM
        for d in range(_DIM):
            uu = plsc.load_gather(uval, [rbase + d])
            ii = plsc.load_gather(ival, [rbase + d])
            acc = acc + uu * ii
            sq = sq + (uu * uu + ii * ii)
        idv = iid_v[pl.ds(g * _L, _L)]
        bias = plsc.load_gather(brun, [ridx * 16 + (idv & 15)])
        lab = lab_v[pl.ds(g * _L, _L)]
        pred = acc + bias
        w = (_A - _B) * lab + _B
        err = lab - pred
        return loss_acc + w * err * err, l2_acc + sq

    loss_vec, l2_vec = lax.fori_loop(
        0, _NGRP,
        body,
        (jnp.zeros((_L,), jnp.float32), jnp.zeros((_L,), jnp.float32)),
    )

    loss_st[...] = loss_vec
    l2_st[...] = 0.5 * l2_vec
    pltpu.sync_copy(loss_st, loss_out.at[pl.ds(wid * _L, _L)])
    pltpu.sync_copy(l2_st, l2_out.at[pl.ds(wid * _L, _L)])


def kernel(user_id, item_id, label, user_table, item_table, item_bias_table):
    loss_p, l2_p = _wrmf_sc(
        user_id.astype(jnp.int32),
        item_id.astype(jnp.int32),
        label,
        user_table.T,
        item_table.T,
        item_bias_table.reshape(-1),
    )
    return jnp.sum(loss_p), jnp.sum(l2_p)


# speedup vs baseline: 15.8526x; 1.0484x over previous
import functools

import jax
import jax.numpy as jnp
from jax import lax
from jax.experimental import pallas as pl
from jax.experimental.pallas import tpu as pltpu
from jax.experimental.pallas import tpu_sc as plsc

_DIM = 32
_BATCH = 16384
_A = 1.0
_B = 1.0

_info = plsc.get_sparse_core_info()
_NC, _NS, _L = _info.num_cores, _info.num_subcores, _info.num_lanes
_NW = _NC * _NS
_BPW = _BATCH // _NW
_NGRP = _BPW // _L

_mesh = plsc.VectorSubcoreMesh(core_axis_name="c", subcore_axis_name="s")


@functools.partial(
    pl.kernel,
    mesh=_mesh,
    compiler_params=pltpu.CompilerParams(needs_layout_passes=False),
    out_type=[
        jax.ShapeDtypeStruct((_NW * _L,), jnp.float32),
        jax.ShapeDtypeStruct((_NW * _L,), jnp.float32),
    ],
    scratch_types=[
        pltpu.VMEM((_BPW,), jnp.int32),
        pltpu.VMEM((_BPW,), jnp.int32),
        pltpu.VMEM((_BPW,), jnp.float32),
        pltpu.VMEM((_L, 8, 128), jnp.float32),
        pltpu.VMEM((_L, 8, 128), jnp.float32),
        pltpu.VMEM((_L, 8, 128), jnp.float32),
        pltpu.VMEM((_BPW * _DIM,), jnp.float32),
        pltpu.VMEM((_BPW * _DIM,), jnp.float32),
        pltpu.VMEM((_BPW * 16,), jnp.float32),
        pltpu.VMEM((_L,), jnp.float32),
        pltpu.VMEM((_L,), jnp.float32),
        pltpu.SemaphoreType.DMA,
        pltpu.SemaphoreType.DMA,
        pltpu.SemaphoreType.DMA,
        pltpu.SemaphoreType.DMA,
    ],
)
def _wrmf_sc(uid_hbm, iid_hbm, lab_hbm, ut_hbm, it_hbm, bt_hbm,
             loss_out, l2_out,
             uid_v, iid_v, lab_v, bundle_a, bundle_b, bundle_c,
             uval, ival, brun,
             loss_st, l2_st, sem_a, sem_b2, sem_c, sem_b):
    wid = lax.axis_index("s") * _NC + lax.axis_index("c")
    base = wid * _BPW

    pltpu.sync_copy(uid_hbm.at[pl.ds(base, _BPW)], uid_v)
    pltpu.sync_copy(iid_hbm.at[pl.ds(base, _BPW)], iid_v)
    pltpu.sync_copy(lab_hbm.at[pl.ds(base, _BPW)], lab_v)

    lane = lax.broadcasted_iota(jnp.int32, (_L,), 0)

    bundles = (bundle_a, bundle_b, bundle_c)
    sems = (sem_a, sem_b2, sem_c)
    _NB = len(bundles)

    def chunk(c, carry):
        uvec = uid_v[pl.ds(c * _L, _L)]
        ivec = iid_v[pl.ds(c * _L, _L)]
        ublk = [pl.multiple_of((uvec[j] >> 7) << 7, 128) for j in range(_L)]
        iblk = [pl.multiple_of((ivec[j] >> 7) << 7, 128) for j in range(_L)]
        ulow = uvec & 127
        ilow = ivec & 127
        ridx = c * _L + lane
        phases = ([(ublk, ut_hbm, uval, ulow, dr) for dr in range(4)]
                  + [(iblk, it_hbm, ival, ilow, dr) for dr in range(4)])

        def issue(p):
            blks, table, _, _, dr = phases[p]
            return [pltpu.async_copy(
                table.at[pl.ds(dr * 8, 8), pl.ds(blks[j], 128)],
                bundles[p % _NB].at[j], sems[p % _NB]) for j in range(_L)]

        cps = [None] * _NB
        cps[0] = issue(0)
        cps[1] = issue(1)
        for p in range(8):
            if p + 2 < 8:
                cps[(p + 2) % _NB] = issue(p + 2)
            for cp in cps[p % _NB]:
                cp.wait()
            _, _, dst, low, dr = phases[p]
            for s in range(8):
                d = dr * 8 + s
                val = plsc.load_gather(
                    bundles[p % _NB],
                    [lane, jnp.full((_L,), s, jnp.int32), low])
                plsc.store_scatter(dst, [ridx * _DIM + d], val)
        run = (ivec >> 4) << 4
        for j in range(_L):
            pltpu.async_copy(bt_hbm.at[pl.ds(pl.multiple_of(run[j], 16), 16)],
                             brun.at[pl.ds((c * _L + j) * 16, 16)], sem_b)
        return carry

    lax.fori_loop(0, _NGRP, chunk, 0)
    pltpu.make_async_copy(lab_hbm.at[pl.ds(0, _BPW * 16)], brun, sem_b).wait()

    def body(g, carry):
        loss_acc, l2_acc = carry
        ridx = g * _L + lane
        acc = jnp.zeros((_L,), jnp.float32)
        sq = jnp.zeros((_L,), jnp.float32)
        rbase = ridx * _DIM
        for d in range(_DIM):
            uu = plsc.load_gather(uval, [rbase + d])
            ii = plsc.load_gather(ival, [rbase + d])
            acc = acc + uu * ii
            sq = sq + (uu * uu + ii * ii)
        idv = iid_v[pl.ds(g * _L, _L)]
        bias = plsc.load_gather(brun, [ridx * 16 + (idv & 15)])
        lab = lab_v[pl.ds(g * _L, _L)]
        pred = acc + bias
        w = (_A - _B) * lab + _B
        err = lab - pred
        return loss_acc + w * err * err, l2_acc + sq

    loss_vec, l2_vec = lax.fori_loop(
        0, _NGRP,
        body,
        (jnp.zeros((_L,), jnp.float32), jnp.zeros((_L,), jnp.float32)),
    )

    loss_st[...] = loss_vec
    l2_st[...] = 0.5 * l2_vec
    pltpu.sync_copy(loss_st, loss_out.at[pl.ds(wid * _L, _L)])
    pltpu.sync_copy(l2_st, l2_out.at[pl.ds(wid * _L, _L)])


def kernel(user_id, item_id, label, user_table, item_table, item_bias_table):
    loss_p, l2_p = _wrmf_sc(
        user_id.astype(jnp.int32),
        item_id.astype(jnp.int32),
        label,
        user_table.T,
        item_table.T,
        item_bias_table.reshape(-1),
    )
    return jnp.sum(loss_p), jnp.sum(l2_p)
